# Initial kernel scaffold; baseline (speedup 1.0000x reference)
import jax
import jax.numpy as jnp
from jax.experimental import pallas as pl


def kernel(x, att, Wz, bz, Lz, lbz, Wr, br, Lr, lbr, Wh, bh, Lh, lbh, Wlin, blin, edge_index):
    raise NotImplementedError("write your pallas kernel here")


# speedup vs baseline: 18.2246x; 18.2246x over previous
import functools

import jax
import jax.numpy as jnp
from jax import lax
from jax.experimental import pallas as pl
from jax.experimental.pallas import tpu as pltpu
from jax.experimental.pallas import tpu_sc as plsc

N = 10000
E = 320000
F_IN = 128
HID = 32
P = 12
K3 = 3 * HID

NC = 2
NS = 16
LANES = 128

CH = (E // NS + LANES - 1) // LANES
EPAD = NS * CH * LANES
CHD = (E + NC * NS * LANES - 1) // (NC * NS * LANES)
EPADD = NC * NS * CHD * LANES

RPT = 632
NPAD = NS * RPT

_mesh = plsc.VectorSubcoreMesh(core_axis_name="c", subcore_axis_name="s")


@functools.partial(
    pl.kernel,
    out_type=jax.ShapeDtypeStruct((NC, NPAD, 16), jnp.float32),
    mesh=_mesh,
    scratch_types=[
        pltpu.VMEM((CHD, LANES), jnp.int32),
        pltpu.VMEM((LANES, 16), jnp.float32),
        pltpu.VMEM((NPAD // NS, 16), jnp.float32),
        pltpu.VMEM_SHARED((NPAD, 16), jnp.float32),
    ],
    compiler_params=pltpu.CompilerParams(use_tc_tiling_on_sc=False),
)
def _sc_deg(dst_hbm, ones_hbm, zero_hbm, out_hbm, idx_v, ones_v, zbuf_v, acc_sh):
    c = lax.axis_index("c")
    s = lax.axis_index("s")
    pltpu.sync_copy(dst_hbm.at[c].at[s], idx_v)
    pltpu.sync_copy(ones_hbm, ones_v)
    pltpu.sync_copy(zero_hbm, zbuf_v)
    zn = NPAD // NS
    pltpu.sync_copy(zbuf_v, acc_sh.at[pl.ds(s * zn, zn)])
    plsc.subcore_barrier()

    def body(j, carry):
        pltpu.sync_copy(ones_v, acc_sh.at[idx_v.at[j]], add=True)
        return carry

    lax.fori_loop(0, CHD, body, 0)
    plsc.subcore_barrier()
    pltpu.sync_copy(acc_sh.at[pl.ds(s * RPT, RPT)],
                    out_hbm.at[c].at[pl.ds(s * RPT, RPT)])


@functools.partial(
    pl.kernel,
    out_type=jax.ShapeDtypeStruct((P, NPAD, K3), jnp.float32),
    mesh=_mesh,
    scratch_types=[
        pltpu.VMEM((CH, LANES), jnp.int32),
        pltpu.VMEM((CH, LANES), jnp.int32),
        pltpu.VMEM((LANES, K3), jnp.float32),
        pltpu.VMEM((RPT // 4, K3), jnp.float32),
        pltpu.SemaphoreType.DMA,
        pltpu.VMEM_SHARED((NPAD, K3), jnp.float32),
    ],
    compiler_params=pltpu.CompilerParams(use_tc_tiling_on_sc=False),
)
def _sc_conv(m_hbm, src_hbm, dst_hbm, out_hbm, src_v, dst_v, gbuf, ibuf, sem,
             acc_sh):
    c = lax.axis_index("c")
    s = lax.axis_index("s")
    pltpu.sync_copy(src_hbm.at[s], src_v)
    pltpu.sync_copy(dst_hbm.at[s], dst_v)
    for k in range(P // NC):
        p = k * NC + c
        q4 = RPT // 4
        for ii in range(4):
            pltpu.sync_copy(m_hbm.at[p].at[pl.ds(s * RPT + ii * q4, q4)], ibuf)
            pltpu.sync_copy(ibuf, acc_sh.at[pl.ds(s * RPT + ii * q4, q4)])
        plsc.subcore_barrier()

        def body(j, carry):
            pltpu.async_copy(m_hbm.at[p].at[src_v.at[j]], gbuf, sem).wait()
            pltpu.sync_copy(gbuf, acc_sh.at[dst_v.at[j]], add=True)
            return carry

        lax.fori_loop(0, CH, body, 0)
        plsc.subcore_barrier()
        pltpu.sync_copy(acc_sh.at[pl.ds(s * RPT, RPT)],
                        out_hbm.at[p].at[pl.ds(s * RPT, RPT)])
        plsc.subcore_barrier()


def _tc_proj_body(xt_ref, degs_ref, vcat_ref, m_ref):
    deg = degs_ref[0, :, 0] + degs_ref[1, :, 0] + 1.0
    dinv = lax.rsqrt(deg)
    m = jnp.dot(xt_ref[0], vcat_ref[...], preferred_element_type=jnp.float32)
    m_ref[0] = m * dinv[:, None]


def _tc_gru_body(s_ref, degs_ref, att_ref, lb_ref, lhb_ref, b2_ref,
                 wlin_ref, blin_ref, out_ref):
    nb = s_ref.shape[1]
    deg = degs_ref[0, :, 0] + degs_ref[1, :, 0] + 1.0
    dinv = lax.rsqrt(deg)[:, None]
    probs = jax.nn.softmax(att_ref[...])
    b2 = b2_ref[...][None, :]
    H = jnp.zeros((nb, HID), jnp.float32)
    acc = jnp.zeros((nb, HID), jnp.float32)
    for p in range(P):
        pre = s_ref[p] * dinv + b2
        HB = jnp.dot(H, lb_ref[...], preferred_element_type=jnp.float32)
        Z = jax.nn.sigmoid(pre[:, 0:HID] + HB[:, 0:HID])
        R = jax.nn.sigmoid(pre[:, HID:2 * HID] + HB[:, HID:2 * HID])
        Ht = jnp.tanh(pre[:, 2 * HID:3 * HID] +
                      jnp.dot(H * R, lhb_ref[...],
                              preferred_element_type=jnp.float32))
        H = Z * H + (1.0 - Z) * Ht
        acc = acc + probs[p] * H
    h = jnp.maximum(acc, 0.0)
    out_ref[...] = (jnp.dot(h, wlin_ref[...],
                            preferred_element_type=jnp.float32)
                    + blin_ref[...][None, :])


def kernel(x, att, Wz, bz, Lz, lbz, Wr, br, Lr, lbr, Wh, bh, Lh, lbh,
           Wlin, blin, edge_index):
    f32 = jnp.float32
    LzT, LzB = Lz[:HID], Lz[HID:]
    LrT, LrB = Lr[:HID], Lr[HID:]
    LhT, LhB = Lh[:HID], Lh[HID:]
    Vcat = jnp.concatenate([Wz @ LzT, Wr @ LrT, Wh @ LhT], axis=1)
    b2 = jnp.concatenate([bz @ LzT + lbz, br @ LrT + lbr, bh @ LhT + lbh])
    LB = jnp.concatenate([LzB, LrB], axis=1)

    xt = jnp.transpose(x, (2, 0, 1))
    xt = jnp.pad(xt, ((0, 0), (0, NPAD - N), (0, 0)))
    src = edge_index[0]
    dst = edge_index[1]
    srcp = jnp.concatenate(
        [src, jnp.zeros((EPAD - E,), jnp.int32)]).reshape(NS, CH, LANES)
    dstp = jnp.concatenate(
        [dst, jnp.full((EPAD - E,), N, jnp.int32)]).reshape(NS, CH, LANES)
    dstd = jnp.concatenate(
        [dst, jnp.full((EPADD - E,), N, jnp.int32)]).reshape(NC, NS, CHD,
                                                             LANES)
    ones16 = jnp.ones((LANES, 16), f32)
    zrows = jnp.zeros((NPAD // NS, 16), f32)

    degs = _sc_deg(dstd, ones16, zrows)

    NB1 = RPT
    m_tab = pl.pallas_call(
        _tc_proj_body,
        grid=(P, NPAD // NB1),
        in_specs=[
            pl.BlockSpec((1, NB1, F_IN), lambda p, i: (p, i, 0)),
            pl.BlockSpec((NC, NB1, 16), lambda p, i: (0, i, 0)),
            pl.BlockSpec((F_IN, K3), lambda p, i: (0, 0)),
        ],
        out_specs=pl.BlockSpec((1, NB1, K3), lambda p, i: (p, i, 0)),
        out_shape=jax.ShapeDtypeStruct((P, NPAD, K3), f32),
    )(xt, degs, Vcat)

    s_tab = _sc_conv(m_tab, srcp, dstp)

    NB2 = RPT
    out = pl.pallas_call(
        _tc_gru_body,
        grid=(NPAD // NB2,),
        in_specs=[
            pl.BlockSpec((P, NB2, K3), lambda i: (0, i, 0)),
            pl.BlockSpec((NC, NB2, 16), lambda i: (0, i, 0)),
            pl.BlockSpec((P,), lambda i: (0,)),
            pl.BlockSpec((HID, 2 * HID), lambda i: (0, 0)),
            pl.BlockSpec((HID, HID), lambda i: (0, 0)),
            pl.BlockSpec((K3,), lambda i: (0,)),
            pl.BlockSpec((HID, P), lambda i: (0, 0)),
            pl.BlockSpec((P,), lambda i: (0,)),
        ],
        out_specs=pl.BlockSpec((NB2, P), lambda i: (i, 0)),
        out_shape=jax.ShapeDtypeStruct((NPAD, P), f32),
    )(s_tab, degs, att, LB, LhB, b2, Wlin, blin)
    return out[:N]
